# baseline (speedup 1.0000x reference)
import jax
import jax.numpy as jnp
from jax.experimental import pallas as pl
from jax.experimental.pallas import tpu as pltpu

_BN = 512
_NFIX = 16
_TAU = 4e-3


def _acc_row_sum(v):
    err = jnp.zeros(v.shape, jnp.float32)
    w = v.shape[1]
    while w > 1:
        h = w // 2
        a = v[:, :h]
        b = v[:, h:w]
        s = a + b
        ap = s - b
        bp = s - ap
        e = (a - ap) + (b - bp)
        err = err[:, :h] + err[:, h:w] + e
        v = s
        w = h
    return v + err


def _split3(v, bf16, f32):
    h = v.astype(bf16)
    l = (v - h.astype(f32)).astype(bf16)
    ll = (v - h.astype(f32) - l.astype(f32)).astype(bf16)
    return h, l, ll


def _tile_kernel(x_ref, c_ref, dist_ref, assign_ref,
                 sh_ref, sl_ref, sll_ref, cn_ref, tri_ref):
    f32 = jnp.float32
    bf16 = jnp.bfloat16
    i32 = jnp.int32

    @pl.when(pl.program_id(0) == 0)
    def _():
        c = c_ref[...]
        kk_ = c.shape[0]
        c2 = jnp.sum(c * c, axis=1, keepdims=True)
        sh, sl, sll = _split3(-2.0 * c, bf16, f32)
        sh_ref[...] = sh
        sl_ref[...] = sl
        sll_ref[...] = sll
        c2h, c2l, c2ll = _split3(c2, bf16, f32)
        cn_ref[...] = jnp.concatenate(
            [c2h, c2l, c2ll,
             jnp.ones((kk_, 3), bf16), jnp.zeros((kk_, 2), bf16)], axis=1)
        n_ = tri_ref.shape[0]
        tri_ref[...] = (jax.lax.broadcasted_iota(i32, (n_, n_), 0) <=
                        jax.lax.broadcasted_iota(i32, (n_, n_), 1)).astype(bf16)

    x = x_ref[...]
    bn_ = x.shape[0]
    x2 = jnp.sum(x * x, axis=1, keepdims=True)
    xh = x.astype(bf16)
    xl = (x - xh.astype(f32)).astype(bf16)
    x2h, x2l, x2ll = _split3(x2, bf16, f32)
    xn = jnp.concatenate(
        [jnp.ones((bn_, 3), bf16), x2h, x2l, x2ll,
         jnp.zeros((bn_, 2), bf16)], axis=1)

    dims = (((1,), (1,)), ((), ()))
    dist = jax.lax.dot_general(sh_ref[...], xh, dims, preferred_element_type=f32)
    dist += jax.lax.dot_general(sh_ref[...], xl, dims, preferred_element_type=f32)
    dist += jax.lax.dot_general(sl_ref[...], xh, dims, preferred_element_type=f32)
    dist += jax.lax.dot_general(cn_ref[...], xn, dims, preferred_element_type=f32)
    dist_ref[...] = dist

    kk, bn = dist.shape
    rows = jax.lax.broadcasted_iota(i32, dist.shape, 0)
    ibig = jnp.int32(jnp.iinfo(jnp.int32).max)
    inf = jnp.float32(jnp.inf)

    d1 = jnp.min(dist, axis=0, keepdims=True)
    arg1 = jnp.min(jnp.where(dist == d1, rows, ibig), axis=0, keepdims=True)
    assign_ref[...] = arg1

    near = (dist < d1 + _TAU).astype(i32)
    ncand = jnp.sum(near, axis=0, keepdims=True)
    flagged = ncand >= 2

    @pl.when(jnp.sum(ncand) > bn)
    def _refine():
        masked = jnp.where(rows == arg1, inf, dist)
        d2 = jnp.min(masked, axis=0, keepdims=True)
        arg2 = jnp.min(jnp.where(masked == d2, rows, ibig),
                       axis=0, keepdims=True)

        flagf = flagged.astype(bf16)
        rank = jax.lax.dot_general(flagf, tri_ref[...],
                                   (((1,), (0,)), ((), ())),
                                   preferred_element_type=f32)
        hots = [flagged & (rank == jnp.float32(m + 1)) for m in range(_NFIX)]
        self32 = jnp.concatenate([h.astype(f32) for h in hots], axis=0)
        selbf = jnp.concatenate([h.astype(bf16) for h in hots], axis=0)

        a1f = jnp.sum(self32 * arg1.astype(f32), axis=1, keepdims=True)
        a2f = jnp.sum(self32 * arg2.astype(f32), axis=1, keepdims=True)

        nd = (((1,), (0,)), ((), ()))
        xll = (x - xh.astype(f32) - xl.astype(f32)).astype(bf16)
        xrow = jax.lax.dot_general(selbf, xh, nd, preferred_element_type=f32)
        xrow += jax.lax.dot_general(selbf, xl, nd, preferred_element_type=f32)
        xrow += jax.lax.dot_general(selbf, xll, nd, preferred_element_type=f32)

        a12 = jnp.concatenate([a1f, a2f], axis=0)
        ohc = (jax.lax.broadcasted_iota(i32, (2 * _NFIX, kk), 1).astype(f32)
               == a12).astype(bf16)
        srow = jax.lax.dot_general(ohc, sh_ref[...], nd, preferred_element_type=f32)
        srow += jax.lax.dot_general(ohc, sl_ref[...], nd, preferred_element_type=f32)
        srow += jax.lax.dot_general(ohc, sll_ref[...], nd, preferred_element_type=f32)
        crow = -0.5 * srow

        dd = jnp.concatenate([xrow, xrow], axis=0) - crow
        tot = _acc_row_sum(dd * dd)
        e1 = tot[:_NFIX]
        e2 = tot[_NFIX:]

        pick2 = (e2 < e1) | ((e2 == e1) & (a2f < a1f))
        fixedf = jnp.where(pick2, a2f, a1f)

        upd = jnp.sum(self32 * fixedf, axis=0, keepdims=True)
        anyhot = jnp.sum(self32, axis=0, keepdims=True) > 0.0
        assign_ref[...] = jnp.where(anyhot, upd.astype(i32), arg1)


def kernel(inputs, centroids):
    n, d = inputs.shape
    k, _ = centroids.shape
    bn = _BN
    dist, assign = pl.pallas_call(
        _tile_kernel,
        grid=(n // bn,),
        in_specs=[
            pl.BlockSpec((bn, d), lambda j: (j, 0)),
            pl.BlockSpec((k, d), lambda j: (0, 0)),
        ],
        out_specs=[
            pl.BlockSpec((k, bn), lambda j: (0, j)),
            pl.BlockSpec((1, bn), lambda j: (0, j)),
        ],
        out_shape=[
            jax.ShapeDtypeStruct((k, n), jnp.float32),
            jax.ShapeDtypeStruct((1, n), jnp.int32),
        ],
        scratch_shapes=[
            pltpu.VMEM((k, d), jnp.bfloat16),
            pltpu.VMEM((k, d), jnp.bfloat16),
            pltpu.VMEM((k, d), jnp.bfloat16),
            pltpu.VMEM((k, 8), jnp.bfloat16),
            pltpu.VMEM((bn, bn), jnp.bfloat16),
        ],
        compiler_params=pltpu.CompilerParams(
            dimension_semantics=("arbitrary",)),
    )(inputs, centroids)
    return dist, assign[0]


# speedup vs baseline: 1.0131x; 1.0131x over previous
import jax
import jax.numpy as jnp
from jax.experimental import pallas as pl
from jax.experimental.pallas import tpu as pltpu

_BN = 512
_NFIX = 16
_TAU = 4e-3


def _acc_row_sum(v):
    err = jnp.zeros(v.shape, jnp.float32)
    w = v.shape[1]
    while w > 1:
        h = w // 2
        a = v[:, :h]
        b = v[:, h:w]
        s = a + b
        ap = s - b
        bp = s - ap
        e = (a - ap) + (b - bp)
        err = err[:, :h] + err[:, h:w] + e
        v = s
        w = h
    return v + err


def _split3(v, bf16, f32):
    h = v.astype(bf16)
    l = (v - h.astype(f32)).astype(bf16)
    ll = (v - h.astype(f32) - l.astype(f32)).astype(bf16)
    return h, l, ll


def _tile_kernel(x_ref, c_ref, dist_ref, assign_ref,
                 s_ref, sll_ref, tri_ref):
    f32 = jnp.float32
    bf16 = jnp.bfloat16
    i32 = jnp.int32
    d_ = c_ref.shape[1]

    @pl.when(pl.program_id(0) == 0)
    def _():
        c = c_ref[...]
        kk_ = c.shape[0]
        c2 = jnp.sum(c * c, axis=1, keepdims=True)
        sh, sl, sll = _split3(-2.0 * c, bf16, f32)
        sll_ref[...] = sll
        c2h, c2l, c2ll = _split3(c2, bf16, f32)
        cn = jnp.concatenate(
            [c2h, c2l, c2ll,
             jnp.ones((kk_, 3), bf16), jnp.zeros((kk_, 2), bf16)], axis=1)
        s_ref[...] = jnp.concatenate([sh, sh, sl, cn], axis=1)
        n_ = tri_ref.shape[0]
        tri_ref[...] = (jax.lax.broadcasted_iota(i32, (n_, n_), 0) <=
                        jax.lax.broadcasted_iota(i32, (n_, n_), 1)).astype(bf16)

    x = x_ref[...]
    bn_ = x.shape[0]
    x2 = jnp.sum(x * x, axis=1, keepdims=True)
    xh = x.astype(bf16)
    xl = (x - xh.astype(f32)).astype(bf16)
    x2h, x2l, x2ll = _split3(x2, bf16, f32)
    xn = jnp.concatenate(
        [jnp.ones((bn_, 3), bf16), x2h, x2l, x2ll,
         jnp.zeros((bn_, 2), bf16)], axis=1)
    xcat = jnp.concatenate([xh, xl, xh, xn], axis=1)

    dims = (((1,), (1,)), ((), ()))
    dist = jax.lax.dot_general(s_ref[...], xcat, dims,
                               preferred_element_type=f32)
    dist_ref[...] = dist

    kk, bn = dist.shape
    rows = jax.lax.broadcasted_iota(i32, dist.shape, 0)
    ibig = jnp.int32(jnp.iinfo(jnp.int32).max)
    inf = jnp.float32(jnp.inf)

    d1 = jnp.min(dist, axis=0, keepdims=True)
    arg1 = jnp.min(jnp.where(dist == d1, rows, ibig), axis=0, keepdims=True)
    assign_ref[...] = arg1

    near = (dist < d1 + _TAU).astype(i32)
    ncand = jnp.sum(near, axis=0, keepdims=True)
    flagged = ncand >= 2

    @pl.when(jnp.sum(ncand) > bn)
    def _refine():
        masked = jnp.where(rows == arg1, inf, dist)
        d2 = jnp.min(masked, axis=0, keepdims=True)
        arg2 = jnp.min(jnp.where(masked == d2, rows, ibig),
                       axis=0, keepdims=True)

        flagf = flagged.astype(bf16)
        rank = jax.lax.dot_general(flagf, tri_ref[...],
                                   (((1,), (0,)), ((), ())),
                                   preferred_element_type=f32)
        hots = [flagged & (rank == jnp.float32(m + 1)) for m in range(_NFIX)]
        self32 = jnp.concatenate([h.astype(f32) for h in hots], axis=0)
        selbf = jnp.concatenate([h.astype(bf16) for h in hots], axis=0)

        a1f = jnp.sum(self32 * arg1.astype(f32), axis=1, keepdims=True)
        a2f = jnp.sum(self32 * arg2.astype(f32), axis=1, keepdims=True)

        nd = (((1,), (0,)), ((), ()))
        xll = (x - xh.astype(f32) - xl.astype(f32)).astype(bf16)
        xrow = jax.lax.dot_general(selbf, xh, nd, preferred_element_type=f32)
        xrow += jax.lax.dot_general(selbf, xl, nd, preferred_element_type=f32)
        xrow += jax.lax.dot_general(selbf, xll, nd, preferred_element_type=f32)

        a12 = jnp.concatenate([a1f, a2f], axis=0)
        ohc = (jax.lax.broadcasted_iota(i32, (2 * _NFIX, kk), 1).astype(f32)
               == a12).astype(bf16)
        srow = jax.lax.dot_general(ohc, s_ref[:, :d_], nd,
                                   preferred_element_type=f32)
        srow += jax.lax.dot_general(ohc, s_ref[:, 2 * d_:3 * d_], nd,
                                    preferred_element_type=f32)
        srow += jax.lax.dot_general(ohc, sll_ref[...], nd, preferred_element_type=f32)
        crow = -0.5 * srow

        dd = jnp.concatenate([xrow, xrow], axis=0) - crow
        tot = _acc_row_sum(dd * dd)
        e1 = tot[:_NFIX]
        e2 = tot[_NFIX:]

        pick2 = (e2 < e1) | ((e2 == e1) & (a2f < a1f))
        fixedf = jnp.where(pick2, a2f, a1f)

        upd = jnp.sum(self32 * fixedf, axis=0, keepdims=True)
        anyhot = jnp.sum(self32, axis=0, keepdims=True) > 0.0
        assign_ref[...] = jnp.where(anyhot, upd.astype(i32), arg1)


def kernel(inputs, centroids):
    n, d = inputs.shape
    k, _ = centroids.shape
    bn = _BN
    dist, assign = pl.pallas_call(
        _tile_kernel,
        grid=(n // bn,),
        in_specs=[
            pl.BlockSpec((bn, d), lambda j: (j, 0)),
            pl.BlockSpec((k, d), lambda j: (0, 0)),
        ],
        out_specs=[
            pl.BlockSpec((k, bn), lambda j: (0, j)),
            pl.BlockSpec((1, bn), lambda j: (0, j)),
        ],
        out_shape=[
            jax.ShapeDtypeStruct((k, n), jnp.float32),
            jax.ShapeDtypeStruct((1, n), jnp.int32),
        ],
        scratch_shapes=[
            pltpu.VMEM((k, 3 * d + 8), jnp.bfloat16),
            pltpu.VMEM((k, d), jnp.bfloat16),
            pltpu.VMEM((bn, bn), jnp.bfloat16),
        ],
        compiler_params=pltpu.CompilerParams(
            dimension_semantics=("arbitrary",)),
    )(inputs, centroids)
    return dist, assign[0]


# speedup vs baseline: 1.0841x; 1.0701x over previous
import jax
import jax.numpy as jnp
from jax.experimental import pallas as pl
from jax.experimental.pallas import tpu as pltpu

_BN = 512
_NCHUNK = 4
_NFIX = 16
_TAU = 4e-3


def _acc_row_sum(v):
    err = jnp.zeros(v.shape, jnp.float32)
    w = v.shape[1]
    while w > 1:
        h = w // 2
        a = v[:, :h]
        b = v[:, h:w]
        s = a + b
        ap = s - b
        bp = s - ap
        e = (a - ap) + (b - bp)
        err = err[:, :h] + err[:, h:w] + e
        v = s
        w = h
    return v + err


def _split3(v, bf16, f32):
    h = v.astype(bf16)
    l = (v - h.astype(f32)).astype(bf16)
    ll = (v - h.astype(f32) - l.astype(f32)).astype(bf16)
    return h, l, ll


def _tile_kernel(x_ref, c_ref, dist_ref, assign_ref,
                 s_ref, sll_ref, tri_ref):
    f32 = jnp.float32
    bf16 = jnp.bfloat16
    i32 = jnp.int32
    d_ = c_ref.shape[1]

    @pl.when(pl.program_id(0) == 0)
    def _():
        c = c_ref[...]
        kk_ = c.shape[0]
        c2 = jnp.sum(c * c, axis=1, keepdims=True)
        sh, sl, sll = _split3(-2.0 * c, bf16, f32)
        sll_ref[...] = sll
        c2h, c2l, c2ll = _split3(c2, bf16, f32)
        cn = jnp.concatenate(
            [c2h, c2l, c2ll,
             jnp.ones((kk_, 3), bf16), jnp.zeros((kk_, 2), bf16)], axis=1)
        s_ref[...] = jnp.concatenate([sh, sh, sl, cn], axis=1)
        n_ = tri_ref.shape[0]
        tri_ref[...] = (jax.lax.broadcasted_iota(i32, (n_, n_), 0) <=
                        jax.lax.broadcasted_iota(i32, (n_, n_), 1)).astype(bf16)

    x = x_ref[...]
    bn_ = x.shape[0]
    x2 = jnp.sum(x * x, axis=1, keepdims=True)
    xh = x.astype(bf16)
    xl = (x - xh.astype(f32)).astype(bf16)
    x2h, x2l, x2ll = _split3(x2, bf16, f32)
    xn = jnp.concatenate(
        [jnp.ones((bn_, 3), bf16), x2h, x2l, x2ll,
         jnp.zeros((bn_, 2), bf16)], axis=1)
    xcat = jnp.concatenate([xh, xl, xh, xn], axis=1)

    kk = s_ref.shape[0]
    ck = kk // _NCHUNK
    ibig = jnp.int32(jnp.iinfo(jnp.int32).max)
    inf = jnp.float32(jnp.inf)
    dims = (((1,), (1,)), ((), ()))
    d1s, args = [], []
    for t in range(_NCHUNK):
        dist_t = jax.lax.dot_general(s_ref[t * ck:(t + 1) * ck, :], xcat,
                                     dims, preferred_element_type=f32)
        dist_ref[t * ck:(t + 1) * ck, :] = dist_t
        rows_t = jax.lax.broadcasted_iota(i32, dist_t.shape, 0) + t * ck
        d1_t = jnp.min(dist_t, axis=0, keepdims=True)
        a1_t = jnp.min(jnp.where(dist_t == d1_t, rows_t, ibig),
                       axis=0, keepdims=True)
        d1s.append(d1_t)
        args.append(a1_t)

    d1 = d1s[0]
    for t in range(1, _NCHUNK):
        d1 = jnp.minimum(d1, d1s[t])
    arg1 = ibig * jnp.ones_like(args[0])
    for t in reversed(range(_NCHUNK)):
        arg1 = jnp.where(d1s[t] == d1, args[t], arg1)
    assign_ref[...] = arg1
    bn = d1.shape[1]

    dist = dist_ref[...]
    rows = jax.lax.broadcasted_iota(i32, dist.shape, 0)
    near = (dist < d1 + _TAU).astype(i32)
    ncand = jnp.sum(near, axis=0, keepdims=True)
    flagged = ncand >= 2

    @pl.when(jnp.sum(ncand) > bn)
    def _refine():
        masked = jnp.where(rows == arg1, inf, dist)
        d2 = jnp.min(masked, axis=0, keepdims=True)
        arg2 = jnp.min(jnp.where(masked == d2, rows, ibig),
                       axis=0, keepdims=True)

        flagf = flagged.astype(bf16)
        rank = jax.lax.dot_general(flagf, tri_ref[...],
                                   (((1,), (0,)), ((), ())),
                                   preferred_element_type=f32)
        hots = [flagged & (rank == jnp.float32(m + 1)) for m in range(_NFIX)]
        self32 = jnp.concatenate([h.astype(f32) for h in hots], axis=0)
        selbf = jnp.concatenate([h.astype(bf16) for h in hots], axis=0)

        a1f = jnp.sum(self32 * arg1.astype(f32), axis=1, keepdims=True)
        a2f = jnp.sum(self32 * arg2.astype(f32), axis=1, keepdims=True)

        nd = (((1,), (0,)), ((), ()))
        xll = (x - xh.astype(f32) - xl.astype(f32)).astype(bf16)
        xrow = jax.lax.dot_general(selbf, xh, nd, preferred_element_type=f32)
        xrow += jax.lax.dot_general(selbf, xl, nd, preferred_element_type=f32)
        xrow += jax.lax.dot_general(selbf, xll, nd, preferred_element_type=f32)

        a12 = jnp.concatenate([a1f, a2f], axis=0)
        ohc = (jax.lax.broadcasted_iota(i32, (2 * _NFIX, kk), 1).astype(f32)
               == a12).astype(bf16)
        srow = jax.lax.dot_general(ohc, s_ref[:, :d_], nd,
                                   preferred_element_type=f32)
        srow += jax.lax.dot_general(ohc, s_ref[:, 2 * d_:3 * d_], nd,
                                    preferred_element_type=f32)
        srow += jax.lax.dot_general(ohc, sll_ref[...], nd, preferred_element_type=f32)
        crow = -0.5 * srow

        dd = jnp.concatenate([xrow, xrow], axis=0) - crow
        tot = _acc_row_sum(dd * dd)
        e1 = tot[:_NFIX]
        e2 = tot[_NFIX:]

        pick2 = (e2 < e1) | ((e2 == e1) & (a2f < a1f))
        fixedf = jnp.where(pick2, a2f, a1f)

        upd = jnp.sum(self32 * fixedf, axis=0, keepdims=True)
        anyhot = jnp.sum(self32, axis=0, keepdims=True) > 0.0
        assign_ref[...] = jnp.where(anyhot, upd.astype(i32), arg1)


def kernel(inputs, centroids):
    n, d = inputs.shape
    k, _ = centroids.shape
    bn = _BN
    dist, assign = pl.pallas_call(
        _tile_kernel,
        grid=(n // bn,),
        in_specs=[
            pl.BlockSpec((bn, d), lambda j: (j, 0)),
            pl.BlockSpec((k, d), lambda j: (0, 0)),
        ],
        out_specs=[
            pl.BlockSpec((k, bn), lambda j: (0, j)),
            pl.BlockSpec((1, bn), lambda j: (0, j)),
        ],
        out_shape=[
            jax.ShapeDtypeStruct((k, n), jnp.float32),
            jax.ShapeDtypeStruct((1, n), jnp.int32),
        ],
        scratch_shapes=[
            pltpu.VMEM((k, 3 * d + 8), jnp.bfloat16),
            pltpu.VMEM((k, d), jnp.bfloat16),
            pltpu.VMEM((bn, bn), jnp.bfloat16),
        ],
        compiler_params=pltpu.CompilerParams(
            dimension_semantics=("arbitrary",)),
    )(inputs, centroids)
    return dist, assign[0]


# speedup vs baseline: 1.1999x; 1.1068x over previous
import jax
import jax.numpy as jnp
from jax.experimental import pallas as pl
from jax.experimental.pallas import tpu as pltpu

_BN = 1024
_NCHUNK = 4
_NFIX = 16
_TAU = 4e-3


def _acc_row_sum(v):
    err = jnp.zeros(v.shape, jnp.float32)
    w = v.shape[1]
    while w > 1:
        h = w // 2
        a = v[:, :h]
        b = v[:, h:w]
        s = a + b
        ap = s - b
        bp = s - ap
        e = (a - ap) + (b - bp)
        err = err[:, :h] + err[:, h:w] + e
        v = s
        w = h
    return v + err


def _split3(v, bf16, f32):
    h = v.astype(bf16)
    l = (v - h.astype(f32)).astype(bf16)
    ll = (v - h.astype(f32) - l.astype(f32)).astype(bf16)
    return h, l, ll


def _tile_kernel(x_ref, c_ref, dist_ref, assign_ref,
                 s_ref, sll_ref, tri_ref):
    f32 = jnp.float32
    bf16 = jnp.bfloat16
    i32 = jnp.int32
    d_ = c_ref.shape[1]

    @pl.when(pl.program_id(0) == 0)
    def _():
        c = c_ref[...]
        kk_ = c.shape[0]
        c2 = jnp.sum(c * c, axis=1, keepdims=True)
        sh, sl, sll = _split3(-2.0 * c, bf16, f32)
        sll_ref[...] = sll
        c2h, c2l, c2ll = _split3(c2, bf16, f32)
        cn = jnp.concatenate(
            [c2h, c2l, c2ll,
             jnp.ones((kk_, 3), bf16), jnp.zeros((kk_, 2), bf16)], axis=1)
        s_ref[...] = jnp.concatenate([sh, sh, sl, cn], axis=1)
        n_ = tri_ref.shape[0]
        tri_ref[...] = (jax.lax.broadcasted_iota(i32, (n_, n_), 0) <=
                        jax.lax.broadcasted_iota(i32, (n_, n_), 1)).astype(bf16)

    x = x_ref[...]
    bn_ = x.shape[0]
    x2 = jnp.sum(x * x, axis=1, keepdims=True)
    xh = x.astype(bf16)
    xl = (x - xh.astype(f32)).astype(bf16)
    x2h, x2l, x2ll = _split3(x2, bf16, f32)
    xn = jnp.concatenate(
        [jnp.ones((bn_, 3), bf16), x2h, x2l, x2ll,
         jnp.zeros((bn_, 2), bf16)], axis=1)
    xcat = jnp.concatenate([xh, xl, xh, xn], axis=1)

    kk = s_ref.shape[0]
    ck = kk // _NCHUNK
    ibig = jnp.int32(jnp.iinfo(jnp.int32).max)
    inf = jnp.float32(jnp.inf)
    dims = (((1,), (1,)), ((), ()))
    d1s, args = [], []
    for t in range(_NCHUNK):
        dist_t = jax.lax.dot_general(s_ref[t * ck:(t + 1) * ck, :], xcat,
                                     dims, preferred_element_type=f32)
        dist_ref[t * ck:(t + 1) * ck, :] = dist_t
        rows_t = jax.lax.broadcasted_iota(i32, dist_t.shape, 0) + t * ck
        d1_t = jnp.min(dist_t, axis=0, keepdims=True)
        a1_t = jnp.min(jnp.where(dist_t == d1_t, rows_t, ibig),
                       axis=0, keepdims=True)
        d1s.append(d1_t)
        args.append(a1_t)

    d1 = d1s[0]
    for t in range(1, _NCHUNK):
        d1 = jnp.minimum(d1, d1s[t])
    arg1 = ibig * jnp.ones_like(args[0])
    for t in reversed(range(_NCHUNK)):
        arg1 = jnp.where(d1s[t] == d1, args[t], arg1)
    assign_ref[...] = arg1
    bn = d1.shape[1]

    dist = dist_ref[...]
    rows = jax.lax.broadcasted_iota(i32, dist.shape, 0)
    near = (dist < d1 + _TAU).astype(i32)
    ncand = jnp.sum(near, axis=0, keepdims=True)
    flagged = ncand >= 2

    @pl.when(jnp.sum(ncand) > bn)
    def _refine():
        masked = jnp.where(rows == arg1, inf, dist)
        d2 = jnp.min(masked, axis=0, keepdims=True)
        arg2 = jnp.min(jnp.where(masked == d2, rows, ibig),
                       axis=0, keepdims=True)

        flagf = flagged.astype(bf16)
        rank = jax.lax.dot_general(flagf, tri_ref[...],
                                   (((1,), (0,)), ((), ())),
                                   preferred_element_type=f32)
        hots = [flagged & (rank == jnp.float32(m + 1)) for m in range(_NFIX)]
        self32 = jnp.concatenate([h.astype(f32) for h in hots], axis=0)
        selbf = jnp.concatenate([h.astype(bf16) for h in hots], axis=0)

        a1f = jnp.sum(self32 * arg1.astype(f32), axis=1, keepdims=True)
        a2f = jnp.sum(self32 * arg2.astype(f32), axis=1, keepdims=True)

        nd = (((1,), (0,)), ((), ()))
        xll = (x - xh.astype(f32) - xl.astype(f32)).astype(bf16)
        xrow = jax.lax.dot_general(selbf, xh, nd, preferred_element_type=f32)
        xrow += jax.lax.dot_general(selbf, xl, nd, preferred_element_type=f32)
        xrow += jax.lax.dot_general(selbf, xll, nd, preferred_element_type=f32)

        a12 = jnp.concatenate([a1f, a2f], axis=0)
        ohc = (jax.lax.broadcasted_iota(i32, (2 * _NFIX, kk), 1).astype(f32)
               == a12).astype(bf16)
        srow = jax.lax.dot_general(ohc, s_ref[:, :d_], nd,
                                   preferred_element_type=f32)
        srow += jax.lax.dot_general(ohc, s_ref[:, 2 * d_:3 * d_], nd,
                                    preferred_element_type=f32)
        srow += jax.lax.dot_general(ohc, sll_ref[...], nd, preferred_element_type=f32)
        crow = -0.5 * srow

        dd = jnp.concatenate([xrow, xrow], axis=0) - crow
        tot = _acc_row_sum(dd * dd)
        e1 = tot[:_NFIX]
        e2 = tot[_NFIX:]

        pick2 = (e2 < e1) | ((e2 == e1) & (a2f < a1f))
        fixedf = jnp.where(pick2, a2f, a1f)

        upd = jnp.sum(self32 * fixedf, axis=0, keepdims=True)
        anyhot = jnp.sum(self32, axis=0, keepdims=True) > 0.0
        assign_ref[...] = jnp.where(anyhot, upd.astype(i32), arg1)


def kernel(inputs, centroids):
    n, d = inputs.shape
    k, _ = centroids.shape
    bn = _BN
    dist, assign = pl.pallas_call(
        _tile_kernel,
        grid=(n // bn,),
        in_specs=[
            pl.BlockSpec((bn, d), lambda j: (j, 0)),
            pl.BlockSpec((k, d), lambda j: (0, 0)),
        ],
        out_specs=[
            pl.BlockSpec((k, bn), lambda j: (0, j)),
            pl.BlockSpec((1, bn), lambda j: (0, j)),
        ],
        out_shape=[
            jax.ShapeDtypeStruct((k, n), jnp.float32),
            jax.ShapeDtypeStruct((1, n), jnp.int32),
        ],
        scratch_shapes=[
            pltpu.VMEM((k, 3 * d + 8), jnp.bfloat16),
            pltpu.VMEM((k, d), jnp.bfloat16),
            pltpu.VMEM((bn, bn), jnp.bfloat16),
        ],
        compiler_params=pltpu.CompilerParams(
            dimension_semantics=("arbitrary",)),
    )(inputs, centroids)
    return dist, assign[0]
